# baseline (speedup 1.0000x reference)
import functools

import jax
import jax.numpy as jnp
from jax.experimental import pallas as pl

_CONFS = [(3, 48, 8, 1, 1024), (48, 96, 8, 1, 1024), (96, 192, 12, 2, 384), (192, 384, 16, 2, 128)]
_JOINT_NUM = 21


def _fps(pts, P):
    B, N, _ = pts.shape
    idxs = jnp.zeros((B, P), dtype=jnp.int32)
    dists = jnp.sum((pts - pts[:, :1, :]) ** 2, axis=-1)

    def body(i, state):
        idxs, dists = state
        nxt = jnp.argmax(dists, axis=1).astype(jnp.int32)
        idxs = idxs.at[:, i].set(nxt)
        sel = pts[jnp.arange(B), nxt]
        d = jnp.sum((pts - sel[:, None, :]) ** 2, axis=-1)
        dists = jnp.minimum(dists, d)
        return (idxs, dists)

    idxs, _ = jax.lax.fori_loop(1, P, body, (idxs, dists))
    return idxs


def _xconv(pts, fts, rep, params, li, K, D):
    B, N, _ = pts.shape
    P = rep.shape[1]
    d2 = jnp.sum((rep[:, :, None, :] - pts[:, None, :, :]) ** 2, axis=-1)
    _, nn_idx = jax.lax.top_k(-jax.lax.stop_gradient(d2), K * D)
    nn_idx = nn_idx[:, :, ::D]
    bidx = jnp.arange(B)[:, None, None]
    nbr_pts = pts[bidx, nn_idx]
    nbr_fts = fts[bidx, nn_idx]
    pts_local = nbr_pts - rep[:, :, None, :]
    g = lambda n: (params["l%d_%s_W" % (li, n)], params["l%d_%s_b" % (li, n)])
    W, b = g("d1")
    fl = jax.nn.elu(pts_local @ W + b)
    W, b = g("d2")
    fl = jax.nn.elu(fl @ W + b)
    fcat = jnp.concatenate([fl, nbr_fts], axis=-1)
    xin = pts_local.reshape(B, P, K * 3)
    W, b = g("t0")
    X = jax.nn.elu(xin @ W + b)
    W, b = g("t1")
    X = jax.nn.elu(X @ W + b)
    W, b = g("t2")
    X = X @ W + b
    X = X.reshape(B, P, K, K)
    fX = jnp.einsum("bpkj,bpjc->bpkc", X, fcat)
    W, b = g("end")
    out = fX.reshape(B, P, -1) @ W + b
    return jax.nn.elu(out)


def _elu(x):
    return jnp.where(x > 0, x, jnp.exp(jnp.minimum(x, 0.0)) - 1.0)


def _head_kernel(B, npts, fts_ref, w1, b1, w2, b2, w3, b3, out_ref):
    f = fts_ref[...]
    h = _elu(jnp.dot(f, w1[...], preferred_element_type=jnp.float32) + b1[...])
    h = _elu(jnp.dot(h, w2[...], preferred_element_type=jnp.float32) + b2[...])
    logits = jnp.dot(h, w3[...], preferred_element_type=jnp.float32) + b3[...]
    out_ref[...] = jnp.mean(logits.reshape(B, npts, logits.shape[-1]), axis=1)


def _head(fts, params):
    B, npts, cin = fts.shape
    dout = _JOINT_NUM * 3
    out = pl.pallas_call(
        functools.partial(_head_kernel, B, npts),
        out_shape=jax.ShapeDtypeStruct((B, dout), jnp.float32),
    )(fts.reshape(B * npts, cin), params["f1_W"], params["f1_b"],
      params["f2_W"], params["f2_b"], params["f3_W"], params["f3_b"])
    return out.reshape(B, _JOINT_NUM, 3)


def kernel(x, params):
    pts = x
    fts = x
    for li, (cin, cout, K, D, P) in enumerate(_CONFS):
        if P >= pts.shape[1]:
            rep = pts
        else:
            idx = _fps(jax.lax.stop_gradient(pts), P)
            rep = pts[jnp.arange(pts.shape[0])[:, None], idx]
        fts = _xconv(pts, fts, rep, params, li, K, D)
        pts = rep
    return _head(fts, params)


# speedup vs baseline: 1.5005x; 1.5005x over previous
import functools

import jax
import jax.numpy as jnp
from jax.experimental import pallas as pl
from jax.experimental.pallas import tpu as pltpu

_CONFS = [(3, 48, 8, 1, 1024), (48, 96, 8, 1, 1024), (96, 192, 12, 2, 384), (192, 384, 16, 2, 128)]
_JOINT_NUM = 21


def _fps_kernel(P, ptsT_ref, rx_ref, ry_ref, rz_ref, dref):
    x = ptsT_ref[0]
    y = ptsT_ref[1]
    z = ptsT_ref[2]
    n_iota = jax.lax.broadcasted_iota(jnp.int32, x.shape, 1)
    dref[...] = (x - x[:, 0:1]) ** 2 + (y - y[:, 0:1]) ** 2 + (z - z[:, 0:1]) ** 2
    rx_ref[0, 0:1, :] = x[:, 0:1].T
    ry_ref[0, 0:1, :] = y[:, 0:1].T
    rz_ref[0, 0:1, :] = z[:, 0:1].T

    def body(i, carry):
        x = ptsT_ref[0]
        y = ptsT_ref[1]
        z = ptsT_ref[2]
        d = dref[...]
        nxt = jnp.argmax(d, axis=1, keepdims=True)
        mask = n_iota == nxt
        selx = jnp.sum(jnp.where(mask, x, 0.0), axis=1, keepdims=True)
        sely = jnp.sum(jnp.where(mask, y, 0.0), axis=1, keepdims=True)
        selz = jnp.sum(jnp.where(mask, z, 0.0), axis=1, keepdims=True)
        rx_ref[0, pl.ds(i, 1), :] = selx.T
        ry_ref[0, pl.ds(i, 1), :] = sely.T
        rz_ref[0, pl.ds(i, 1), :] = selz.T
        dd = (x - selx) ** 2 + (y - sely) ** 2 + (z - selz) ** 2
        dref[...] = jnp.minimum(d, dd)
        return carry

    jax.lax.fori_loop(1, P, body, 0)


def _fps_rep(pts, P):
    B, N, _ = pts.shape
    NC = 2
    B2 = B // NC
    ptsT = jnp.transpose(pts, (2, 0, 1))
    outs = pl.pallas_call(
        functools.partial(_fps_kernel, P),
        grid=(NC,),
        in_specs=[pl.BlockSpec((3, B2, N), lambda c: (0, c, 0))],
        out_specs=[pl.BlockSpec((1, P, B2), lambda c: (c, 0, 0))] * 3,
        out_shape=[jax.ShapeDtypeStruct((NC, P, B2), jnp.float32)] * 3,
        scratch_shapes=[pltpu.VMEM((B2, N), jnp.float32)],
        compiler_params=pltpu.CompilerParams(
            dimension_semantics=("parallel",)),
    )(ptsT)
    rx, ry, rz = (jnp.transpose(o, (1, 0, 2)).reshape(P, B).T for o in outs)
    return jnp.stack([rx, ry, rz], axis=-1)


def _xconv(pts, fts, rep, params, li, K, D):
    B, N, _ = pts.shape
    P = rep.shape[1]
    d2 = jnp.sum((rep[:, :, None, :] - pts[:, None, :, :]) ** 2, axis=-1)
    _, nn_idx = jax.lax.top_k(-jax.lax.stop_gradient(d2), K * D)
    nn_idx = nn_idx[:, :, ::D]
    bidx = jnp.arange(B)[:, None, None]
    nbr_pts = pts[bidx, nn_idx]
    nbr_fts = fts[bidx, nn_idx]
    pts_local = nbr_pts - rep[:, :, None, :]
    g = lambda n: (params["l%d_%s_W" % (li, n)], params["l%d_%s_b" % (li, n)])
    W, b = g("d1")
    fl = jax.nn.elu(pts_local @ W + b)
    W, b = g("d2")
    fl = jax.nn.elu(fl @ W + b)
    fcat = jnp.concatenate([fl, nbr_fts], axis=-1)
    xin = pts_local.reshape(B, P, K * 3)
    W, b = g("t0")
    X = jax.nn.elu(xin @ W + b)
    W, b = g("t1")
    X = jax.nn.elu(X @ W + b)
    W, b = g("t2")
    X = X @ W + b
    X = X.reshape(B, P, K, K)
    fX = jnp.einsum("bpkj,bpjc->bpkc", X, fcat)
    W, b = g("end")
    out = fX.reshape(B, P, -1) @ W + b
    return jax.nn.elu(out)


def _elu(x):
    return jnp.where(x > 0, x, jnp.exp(jnp.minimum(x, 0.0)) - 1.0)


def _head_kernel(B, npts, fts_ref, w1, b1, w2, b2, w3, b3, out_ref):
    f = fts_ref[...]
    h = _elu(jnp.dot(f, w1[...], preferred_element_type=jnp.float32) + b1[...])
    h = _elu(jnp.dot(h, w2[...], preferred_element_type=jnp.float32) + b2[...])
    logits = jnp.dot(h, w3[...], preferred_element_type=jnp.float32) + b3[...]
    out_ref[...] = jnp.mean(logits.reshape(B, npts, logits.shape[-1]), axis=1)


def _head(fts, params):
    B, npts, cin = fts.shape
    dout = _JOINT_NUM * 3
    out = pl.pallas_call(
        functools.partial(_head_kernel, B, npts),
        out_shape=jax.ShapeDtypeStruct((B, dout), jnp.float32),
    )(fts.reshape(B * npts, cin), params["f1_W"], params["f1_b"],
      params["f2_W"], params["f2_b"], params["f3_W"], params["f3_b"])
    return out.reshape(B, _JOINT_NUM, 3)


def kernel(x, params):
    pts = x
    fts = x
    for li, (cin, cout, K, D, P) in enumerate(_CONFS):
        if P >= pts.shape[1]:
            rep = pts
        else:
            rep = _fps_rep(pts, P)
        fts = _xconv(pts, fts, rep, params, li, K, D)
        pts = rep
    return _head(fts, params)


# speedup vs baseline: 3.2756x; 2.1830x over previous
import functools

import jax
import jax.numpy as jnp
from jax.experimental import pallas as pl
from jax.experimental.pallas import tpu as pltpu

_CONFS = [(3, 48, 8, 1, 1024), (48, 96, 8, 1, 1024), (96, 192, 12, 2, 384), (192, 384, 16, 2, 128)]
_JOINT_NUM = 21


def _fps_kernel(P, ptsT_ref, rx_ref, ry_ref, rz_ref, dref):
    x = ptsT_ref[0]
    y = ptsT_ref[1]
    z = ptsT_ref[2]
    n_iota = jax.lax.broadcasted_iota(jnp.int32, x.shape, 1)
    dref[...] = (x - x[:, 0:1]) ** 2 + (y - y[:, 0:1]) ** 2 + (z - z[:, 0:1]) ** 2
    rx_ref[0, 0:1, :] = x[:, 0:1].T
    ry_ref[0, 0:1, :] = y[:, 0:1].T
    rz_ref[0, 0:1, :] = z[:, 0:1].T

    def body(i, carry):
        x = ptsT_ref[0]
        y = ptsT_ref[1]
        z = ptsT_ref[2]
        d = dref[...]
        nxt = jnp.argmax(d, axis=1, keepdims=True)
        mask = n_iota == nxt
        selx = jnp.sum(jnp.where(mask, x, 0.0), axis=1, keepdims=True)
        sely = jnp.sum(jnp.where(mask, y, 0.0), axis=1, keepdims=True)
        selz = jnp.sum(jnp.where(mask, z, 0.0), axis=1, keepdims=True)
        rx_ref[0, pl.ds(i, 1), :] = selx.T
        ry_ref[0, pl.ds(i, 1), :] = sely.T
        rz_ref[0, pl.ds(i, 1), :] = selz.T
        dd = (x - selx) ** 2 + (y - sely) ** 2 + (z - selz) ** 2
        dref[...] = jnp.minimum(d, dd)
        return carry

    jax.lax.fori_loop(1, P, body, 0)


def _fps_rep(pts, P):
    B, N, _ = pts.shape
    NC = 2
    B2 = B // NC
    ptsT = jnp.transpose(pts, (2, 0, 1))
    outs = pl.pallas_call(
        functools.partial(_fps_kernel, P),
        grid=(NC,),
        in_specs=[pl.BlockSpec((3, B2, N), lambda c: (0, c, 0))],
        out_specs=[pl.BlockSpec((1, P, B2), lambda c: (c, 0, 0))] * 3,
        out_shape=[jax.ShapeDtypeStruct((NC, P, B2), jnp.float32)] * 3,
        scratch_shapes=[pltpu.VMEM((B2, N), jnp.float32)],
        compiler_params=pltpu.CompilerParams(
            dimension_semantics=("parallel",)),
    )(ptsT)
    rx, ry, rz = (jnp.transpose(o, (1, 0, 2)).reshape(P, B).T for o in outs)
    return jnp.stack([rx, ry, rz], axis=-1)


def _knn_kernel(K, D, ptsT_ref, rep_ref, idx_ref, npx_ref, npy_ref, npz_ref, dref):
    N = ptsT_ref.shape[2]
    P = rep_ref.shape[1]
    px = ptsT_ref[0, 0:1, :]
    py = ptsT_ref[0, 1:2, :]
    pz = ptsT_ref[0, 2:3, :]
    rx = rep_ref[0, :, 0:1]
    ry = rep_ref[0, :, 1:2]
    rz = rep_ref[0, :, 2:3]
    dref[...] = (rx - px) ** 2 + (ry - py) ** 2 + (rz - pz) ** 2
    iota = jax.lax.broadcasted_iota(jnp.int32, (P, N), 1)
    pxb = jnp.broadcast_to(px, (P, N))
    pyb = jnp.broadcast_to(py, (P, N))
    pzb = jnp.broadcast_to(pz, (P, N))
    for j in range(K * D):
        d = dref[...]
        m = jnp.min(d, axis=1, keepdims=True)
        amin = jnp.min(jnp.where(d == m, iota, N), axis=1, keepdims=True)
        sel = iota == amin
        if j % D == 0:
            jj = j // D
            idx_ref[0, :, jj:jj + 1] = amin
            npx_ref[0, :, jj:jj + 1] = jnp.sum(jnp.where(sel, pxb, 0.0), axis=1, keepdims=True)
            npy_ref[0, :, jj:jj + 1] = jnp.sum(jnp.where(sel, pyb, 0.0), axis=1, keepdims=True)
            npz_ref[0, :, jj:jj + 1] = jnp.sum(jnp.where(sel, pzb, 0.0), axis=1, keepdims=True)
        if j != K * D - 1:
            dref[...] = jnp.where(sel, jnp.float32(jnp.inf), d)


def _knn(pts, rep, K, D):
    B, N, _ = pts.shape
    P = rep.shape[1]
    NC = 2
    B2 = B // NC
    ptsT = jnp.transpose(pts, (0, 2, 1))
    outs = pl.pallas_call(
        functools.partial(_knn_kernel, K, D),
        grid=(NC, B2),
        in_specs=[
            pl.BlockSpec((1, 3, N), lambda c, i: (c * (B // NC) + i, 0, 0)),
            pl.BlockSpec((1, P, 3), lambda c, i: (c * (B // NC) + i, 0, 0)),
        ],
        out_specs=[pl.BlockSpec((1, P, K), lambda c, i: (c * (B // NC) + i, 0, 0))] * 4,
        out_shape=[jax.ShapeDtypeStruct((B, P, K), jnp.int32)]
        + [jax.ShapeDtypeStruct((B, P, K), jnp.float32)] * 3,
        scratch_shapes=[pltpu.VMEM((P, N), jnp.float32)],
        compiler_params=pltpu.CompilerParams(
            dimension_semantics=("parallel", "arbitrary")),
    )(ptsT, rep)
    nn_idx = outs[0]
    nbr_pts = jnp.stack(outs[1:], axis=-1)
    return nn_idx, nbr_pts


def _xconv(pts, fts, rep, params, li, K, D):
    B, N, _ = pts.shape
    P = rep.shape[1]
    nn_idx, nbr_pts = _knn(pts, rep, K, D)
    bidx = jnp.arange(B)[:, None, None]
    nbr_fts = fts[bidx, nn_idx]
    pts_local = nbr_pts - rep[:, :, None, :]
    g = lambda n: (params["l%d_%s_W" % (li, n)], params["l%d_%s_b" % (li, n)])
    W, b = g("d1")
    fl = jax.nn.elu(pts_local @ W + b)
    W, b = g("d2")
    fl = jax.nn.elu(fl @ W + b)
    fcat = jnp.concatenate([fl, nbr_fts], axis=-1)
    xin = pts_local.reshape(B, P, K * 3)
    W, b = g("t0")
    X = jax.nn.elu(xin @ W + b)
    W, b = g("t1")
    X = jax.nn.elu(X @ W + b)
    W, b = g("t2")
    X = X @ W + b
    X = X.reshape(B, P, K, K)
    fX = jnp.einsum("bpkj,bpjc->bpkc", X, fcat)
    W, b = g("end")
    out = fX.reshape(B, P, -1) @ W + b
    return jax.nn.elu(out)


def _elu(x):
    return jnp.where(x > 0, x, jnp.exp(jnp.minimum(x, 0.0)) - 1.0)


def _head_kernel(B, npts, fts_ref, w1, b1, w2, b2, w3, b3, out_ref):
    f = fts_ref[...]
    h = _elu(jnp.dot(f, w1[...], preferred_element_type=jnp.float32) + b1[...])
    h = _elu(jnp.dot(h, w2[...], preferred_element_type=jnp.float32) + b2[...])
    logits = jnp.dot(h, w3[...], preferred_element_type=jnp.float32) + b3[...]
    out_ref[...] = jnp.mean(logits.reshape(B, npts, logits.shape[-1]), axis=1)


def _head(fts, params):
    B, npts, cin = fts.shape
    dout = _JOINT_NUM * 3
    out = pl.pallas_call(
        functools.partial(_head_kernel, B, npts),
        out_shape=jax.ShapeDtypeStruct((B, dout), jnp.float32),
    )(fts.reshape(B * npts, cin), params["f1_W"], params["f1_b"],
      params["f2_W"], params["f2_b"], params["f3_W"], params["f3_b"])
    return out.reshape(B, _JOINT_NUM, 3)


def kernel(x, params):
    pts = x
    fts = x
    for li, (cin, cout, K, D, P) in enumerate(_CONFS):
        if P >= pts.shape[1]:
            rep = pts
        else:
            rep = _fps_rep(pts, P)
        fts = _xconv(pts, fts, rep, params, li, K, D)
        pts = rep
    return _head(fts, params)
